# baseline (speedup 1.0000x reference)
import jax
import jax.numpy as jnp
from jax import lax
from jax.experimental import pallas as pl
from jax.experimental.pallas import tpu as pltpu
from jax.experimental.pallas import tpu_sc as plsc

N_NODES = 10000
N_EDGES = 320000
IN_CH = 128
HID_CH = 256
OUT_CH = 128

NPAD = 10240
NC, NS = 2, 16
NW = NC * NS
E_TILE = N_EDGES // NW
CHUNK = 80
NPASS = 5
CPP = 25
NCHUNK = NPASS * CPP
ROWS_TILE = NPAD // NS
ZROWS = 16
MBLK = 1024
GRID_M = NPAD // MBLK

NBUF = 3


def _sc_mesh():
    return plsc.VectorSubcoreMesh(core_axis_name="c", subcore_axis_name="s")


def _deg_body(dst_hbm, out_hbm, dst_v, hist_v):
    c = lax.axis_index("c")
    s = lax.axis_index("s")
    wid = c * NS + s
    pltpu.sync_copy(dst_hbm.at[wid], dst_v)
    zeros = jnp.zeros((16,), jnp.float32)

    def zloop(i, carry):
        hist_v[pl.ds(i * 16, 16)] = zeros
        return carry

    lax.fori_loop(0, NPAD // 16, zloop, 0)
    ones = jnp.ones((16,), jnp.float32)

    def eloop(i, carry):
        idx = dst_v[pl.ds(i * 16, 16)]
        plsc.addupdate_scatter(hist_v, [idx], ones)
        return carry

    lax.fori_loop(0, E_TILE // 16, eloop, 0)
    pltpu.sync_copy(hist_v, out_hbm.at[wid])


def _deg_partials(dst_tiles):
    return pl.kernel(
        _deg_body,
        out_type=jax.ShapeDtypeStruct((NW, NPAD), jnp.float32),
        mesh=_sc_mesh(),
        scratch_types=[
            pltpu.VMEM((E_TILE,), jnp.int32),
            pltpu.VMEM((NPAD,), jnp.float32),
        ],
        compiler_params=pltpu.CompilerParams(needs_layout_passes=False),
    )(dst_tiles)


def _agg_body(y_hbm, src_hbm, dst_hbm, out_hbm, src_v, dst_v, rows_v, zb_v,
              acc_sh, gsem, ssem, isem):
    c = lax.axis_index("c")
    s = lax.axis_index("s")
    wid = c * NS + s

    def load_idx(p):
        sl = p % 2
        pltpu.async_copy(src_hbm.at[wid, p], src_v.at[sl], isem)
        pltpu.async_copy(dst_hbm.at[wid, p], dst_v.at[sl], isem)

    def wait_idx(p):
        sl = p % 2
        pltpu.make_async_copy(src_hbm.at[wid, p], src_v.at[sl], isem).wait()
        pltpu.make_async_copy(dst_hbm.at[wid, p], dst_v.at[sl], isem).wait()

    load_idx(0)

    zeros = jnp.zeros((16,), jnp.float32)

    def zb(i, carry):
        zb_v[i // 8, pl.ds((i % 8) * 16, 16)] = zeros
        return carry

    lax.fori_loop(0, ZROWS * 8, zb, 0)
    base = s * ROWS_TILE

    def zspray(j, carry):
        pltpu.sync_copy(zb_v, acc_sh.at[pl.ds(base + j * ZROWS, ZROWS)])
        return carry

    lax.fori_loop(0, ROWS_TILE // ZROWS, zspray, 0)
    plsc.subcore_barrier()
    wait_idx(0)

    def start_gather(p, j):
        g = p * CPP + j
        pltpu.async_copy(y_hbm.at[src_v.at[p % 2, j]],
                         rows_v.at[g % NBUF], gsem)

    def wait_gather(p, j):
        g = p * CPP + j
        pltpu.make_async_copy(y_hbm.at[src_v.at[p % 2, j]],
                              rows_v.at[g % NBUF], gsem).wait()

    def start_scatter(p, j):
        g = p * CPP + j
        pltpu.async_copy(rows_v.at[g % NBUF], acc_sh.at[dst_v.at[p % 2, j]],
                         ssem, add=True)

    def wait_one_scatter():
        pltpu.make_async_copy(rows_v.at[0], acc_sh.at[dst_v.at[0, 0]],
                              ssem).wait()

    for p in range(NPASS):
        if p > 0:
            wait_one_scatter()
        start_gather(p, 0)

        def chunk(j, carry, p=p):
            wait_gather(p, j)
            start_scatter(p, j)
            if p + 1 < NPASS:
                @pl.when(j == 2)
                def _():
                    load_idx(p + 1)

            @pl.when(j + 1 < CPP)
            def _():
                if p == 0:
                    @pl.when(j >= 1)
                    def _():
                        wait_one_scatter()
                else:
                    wait_one_scatter()
                start_gather(p, j + 1)

            return carry

        lax.fori_loop(0, CPP, chunk, 0)
        if p + 1 < NPASS:
            wait_idx(p + 1)

    for _ in range(2):
        wait_one_scatter()
    plsc.subcore_barrier()
    pltpu.sync_copy(acc_sh.at[pl.ds(base, ROWS_TILE)],
                    out_hbm.at[pl.ds(c * NPAD + base, ROWS_TILE)])


def _edge_aggregate(y, src_tiles, dst_tiles):
    return pl.kernel(
        _agg_body,
        out_type=jax.ShapeDtypeStruct((NC * NPAD, IN_CH), jnp.float32),
        mesh=_sc_mesh(),
        scratch_types=[
            pltpu.VMEM((2, CPP, CHUNK), jnp.int32),
            pltpu.VMEM((2, CPP, CHUNK), jnp.int32),
            pltpu.VMEM((NBUF, CHUNK, IN_CH), jnp.float32),
            pltpu.VMEM((ZROWS, IN_CH), jnp.float32),
            pltpu.VMEM_SHARED((NPAD, IN_CH), jnp.float32),
            pltpu.SemaphoreType.DMA,
            pltpu.SemaphoreType.DMA,
            pltpu.SemaphoreType.DMA,
        ],
        compiler_params=pltpu.CompilerParams(needs_layout_passes=False),
    )(y, src_tiles, dst_tiles)


def _dinv_y_body(hist_ref, x_ref, dinv_ref, y_ref):
    deg = jnp.sum(hist_ref[...], axis=0, keepdims=True) + 1.0
    dinv = lax.rsqrt(deg)
    dinv_t = jnp.reshape(dinv, (MBLK, 1))
    dinv_ref[...] = dinv_t
    y_ref[...] = x_ref[...] * dinv_t


def _dinv_and_y(hist, x_pad):
    return pl.pallas_call(
        _dinv_y_body,
        grid=(GRID_M,),
        in_specs=[
            pl.BlockSpec((NW, MBLK), lambda i: (0, i)),
            pl.BlockSpec((MBLK, IN_CH), lambda i: (i, 0)),
        ],
        out_specs=[
            pl.BlockSpec((MBLK, 1), lambda i: (i, 0)),
            pl.BlockSpec((MBLK, IN_CH), lambda i: (i, 0)),
        ],
        out_shape=[
            jax.ShapeDtypeStruct((NPAD, 1), jnp.float32),
            jax.ShapeDtypeStruct((NPAD, IN_CH), jnp.float32),
        ],
    )(hist, x_pad)


def _mm_body(acc0_ref, acc1_ref, y_ref, dinv_ref, w1_ref, b1_ref, w2_ref,
             y2_ref):
    agg = (acc0_ref[...] + acc1_ref[...] + y_ref[...]) * dinv_ref[...]
    h = jnp.dot(agg, w1_ref[...], preferred_element_type=jnp.float32)
    h = jnp.maximum(h + b1_ref[...], 0.0)
    hw = jnp.dot(h, w2_ref[...], preferred_element_type=jnp.float32)
    y2_ref[...] = hw * dinv_ref[...]


def _both_mm(acc0, acc1, y1, dinv, W1, b1, W2):
    return pl.pallas_call(
        _mm_body,
        grid=(GRID_M,),
        in_specs=[
            pl.BlockSpec((MBLK, IN_CH), lambda i: (i, 0)),
            pl.BlockSpec((MBLK, IN_CH), lambda i: (i, 0)),
            pl.BlockSpec((MBLK, IN_CH), lambda i: (i, 0)),
            pl.BlockSpec((MBLK, 1), lambda i: (i, 0)),
            pl.BlockSpec((IN_CH, HID_CH), lambda i: (0, 0)),
            pl.BlockSpec((1, HID_CH), lambda i: (0, 0)),
            pl.BlockSpec((HID_CH, OUT_CH), lambda i: (0, 0)),
        ],
        out_specs=pl.BlockSpec((MBLK, OUT_CH), lambda i: (i, 0)),
        out_shape=jax.ShapeDtypeStruct((NPAD, OUT_CH), jnp.float32),
    )(acc0, acc1, y1, dinv, W1, b1, W2)


def _final_body(acc0_ref, acc1_ref, y2_ref, dinv_ref, b_ref, out_ref):
    agg = (acc0_ref[...] + acc1_ref[...] + y2_ref[...]) * dinv_ref[...]
    out_ref[...] = jnp.maximum(agg + b_ref[...], 0.0)


def _final_layer(acc0, acc1, y2, dinv, b2):
    return pl.pallas_call(
        _final_body,
        grid=(GRID_M,),
        in_specs=[
            pl.BlockSpec((MBLK, OUT_CH), lambda i: (i, 0)),
            pl.BlockSpec((MBLK, OUT_CH), lambda i: (i, 0)),
            pl.BlockSpec((MBLK, OUT_CH), lambda i: (i, 0)),
            pl.BlockSpec((MBLK, 1), lambda i: (i, 0)),
            pl.BlockSpec((1, OUT_CH), lambda i: (0, 0)),
        ],
        out_specs=pl.BlockSpec((MBLK, OUT_CH), lambda i: (i, 0)),
        out_shape=jax.ShapeDtypeStruct((NPAD, OUT_CH), jnp.float32),
    )(acc0, acc1, y2, dinv, b2)


def kernel(x, edge_index, W1, b1, W2, b2):
    src = edge_index[0].astype(jnp.int32)
    dst = edge_index[1].astype(jnp.int32)
    src_p = src.reshape(NW, NPASS, CPP, CHUNK)
    dst_p = dst.reshape(NW, NPASS, CPP, CHUNK)
    dst_flat_tiles = dst.reshape(NW, E_TILE)
    x_pad = jnp.pad(x, ((0, NPAD - N_NODES), (0, 0)))
    b1r = b1.reshape(1, HID_CH)
    b2r = b2.reshape(1, OUT_CH)

    hist = _deg_partials(dst_flat_tiles)
    dinv, y1 = _dinv_and_y(hist, x_pad)

    acc1 = _edge_aggregate(y1, src_p, dst_p)
    y2 = _both_mm(acc1[:NPAD], acc1[NPAD:], y1, dinv, W1, b1r, W2)
    acc2 = _edge_aggregate(y2, src_p, dst_p)
    out = _final_layer(acc2[:NPAD], acc2[NPAD:], y2, dinv, b2r)
    return out[:N_NODES]


# speedup vs baseline: 1.0360x; 1.0360x over previous
import jax
import jax.numpy as jnp
from jax import lax
from jax.experimental import pallas as pl
from jax.experimental.pallas import tpu as pltpu
from jax.experimental.pallas import tpu_sc as plsc

N_NODES = 10000
N_EDGES = 320000
IN_CH = 128
HID_CH = 256
OUT_CH = 128

NPAD = 10240
NC, NS = 2, 16
NW = NC * NS
E_TILE = N_EDGES // NW
CHUNK = 80
NPASS = 5
CPP = 25
NCHUNK = NPASS * CPP
ROWS_TILE = NPAD // NS
ZROWS = 16
MBLK = 1024
GRID_M = NPAD // MBLK

NBUF = 3


def _sc_mesh():
    return plsc.VectorSubcoreMesh(core_axis_name="c", subcore_axis_name="s")


def _deg_body(dst_hbm, out_hbm, dst_v, hist_v):
    c = lax.axis_index("c")
    s = lax.axis_index("s")
    wid = c * NS + s
    pltpu.sync_copy(dst_hbm.at[wid], dst_v)
    zeros = jnp.zeros((16,), jnp.float32)

    def zloop(i, carry):
        hist_v[pl.ds(i * 16, 16)] = zeros
        return carry

    lax.fori_loop(0, NPAD // 16, zloop, 0)
    ones = jnp.ones((16,), jnp.float32)

    def eloop(i, carry):
        idx = dst_v[pl.ds(i * 16, 16)]
        plsc.addupdate_scatter(hist_v, [idx], ones)
        return carry

    lax.fori_loop(0, E_TILE // 16, eloop, 0)
    pltpu.sync_copy(hist_v, out_hbm.at[wid])


def _deg_partials(dst_tiles):
    return pl.kernel(
        _deg_body,
        out_type=jax.ShapeDtypeStruct((NW, NPAD), jnp.float32),
        mesh=_sc_mesh(),
        scratch_types=[
            pltpu.VMEM((E_TILE,), jnp.int32),
            pltpu.VMEM((NPAD,), jnp.float32),
        ],
        compiler_params=pltpu.CompilerParams(needs_layout_passes=False),
    )(dst_tiles)


def _agg_body(y_hbm, src_hbm, dst_hbm, out_hbm, src_v, dst_v, rows_v, zb_v,
              acc_sh, gsem, ssem, isem):
    c = lax.axis_index("c")
    s = lax.axis_index("s")
    wid = c * NS + s

    def load_idx(p):
        sl = p % 2
        pltpu.async_copy(src_hbm.at[wid, p], src_v.at[sl], isem)
        pltpu.async_copy(dst_hbm.at[wid, p], dst_v.at[sl], isem)

    def wait_idx(p):
        sl = p % 2
        pltpu.make_async_copy(src_hbm.at[wid, p], src_v.at[sl], isem).wait()
        pltpu.make_async_copy(dst_hbm.at[wid, p], dst_v.at[sl], isem).wait()

    load_idx(0)

    zeros = jnp.zeros((16,), jnp.float32)

    def zb(i, carry):
        zb_v[i // 8, pl.ds((i % 8) * 16, 16)] = zeros
        return carry

    lax.fori_loop(0, ZROWS * 8, zb, 0)
    base = s * ROWS_TILE

    def zspray(j, carry):
        pltpu.sync_copy(zb_v, acc_sh.at[pl.ds(base + j * ZROWS, ZROWS)])
        return carry

    lax.fori_loop(0, ROWS_TILE // ZROWS, zspray, 0)
    plsc.subcore_barrier()
    wait_idx(0)

    def start_gather(p, j):
        g = p * CPP + j
        pltpu.async_copy(y_hbm.at[src_v.at[p % 2, j]],
                         rows_v.at[g % NBUF], gsem)

    def wait_gather(p, j):
        g = p * CPP + j
        pltpu.make_async_copy(y_hbm.at[src_v.at[p % 2, j]],
                              rows_v.at[g % NBUF], gsem).wait()

    def start_scatter(p, j):
        g = p * CPP + j
        pltpu.async_copy(rows_v.at[g % NBUF], acc_sh.at[dst_v.at[p % 2, j]],
                         ssem, add=True)

    def wait_one_scatter():
        pltpu.make_async_copy(rows_v.at[0], acc_sh.at[dst_v.at[0, 0]],
                              ssem).wait()

    for p in range(NPASS):
        if p > 0:
            wait_one_scatter()
        start_gather(p, 0)

        def chunk(j, carry, p=p):
            wait_gather(p, j)
            start_scatter(p, j)
            if p + 1 < NPASS:
                @pl.when(j == 2)
                def _():
                    load_idx(p + 1)

            @pl.when(j + 1 < CPP)
            def _():
                if p == 0:
                    @pl.when(j >= 1)
                    def _():
                        wait_one_scatter()
                else:
                    wait_one_scatter()
                start_gather(p, j + 1)

            return carry

        lax.fori_loop(0, CPP, chunk, 0)
        if p + 1 < NPASS:
            wait_idx(p + 1)

    for _ in range(2):
        wait_one_scatter()
    plsc.subcore_barrier()
    pltpu.sync_copy(acc_sh.at[pl.ds(base, ROWS_TILE)],
                    out_hbm.at[pl.ds(c * NPAD + base, ROWS_TILE)])


def _edge_aggregate(y, src_tiles, dst_tiles):
    return pl.kernel(
        _agg_body,
        out_type=jax.ShapeDtypeStruct((NC * NPAD, IN_CH), jnp.float32),
        mesh=_sc_mesh(),
        scratch_types=[
            pltpu.VMEM((2, CPP, CHUNK), jnp.int32),
            pltpu.VMEM((2, CPP, CHUNK), jnp.int32),
            pltpu.VMEM((NBUF, CHUNK, IN_CH), jnp.float32),
            pltpu.VMEM((ZROWS, IN_CH), jnp.float32),
            pltpu.VMEM_SHARED((NPAD, IN_CH), jnp.float32),
            pltpu.SemaphoreType.DMA,
            pltpu.SemaphoreType.DMA,
            pltpu.SemaphoreType.DMA,
        ],
        compiler_params=pltpu.CompilerParams(needs_layout_passes=False),
    )(y, src_tiles, dst_tiles)


def _dinv_y_body(hist_ref, x_ref, dinv_ref, y_ref):
    deg = jnp.sum(hist_ref[...], axis=0, keepdims=True) + 1.0
    dinv = lax.rsqrt(deg)
    dinv_t = jnp.reshape(dinv, (MBLK, 1))
    dinv_ref[...] = dinv_t
    y_ref[...] = x_ref[...] * dinv_t


def _dinv_and_y(hist, x_pad):
    return pl.pallas_call(
        _dinv_y_body,
        grid=(GRID_M,),
        in_specs=[
            pl.BlockSpec((NW, MBLK), lambda i: (0, i)),
            pl.BlockSpec((MBLK, IN_CH), lambda i: (i, 0)),
        ],
        out_specs=[
            pl.BlockSpec((MBLK, 1), lambda i: (i, 0)),
            pl.BlockSpec((MBLK, IN_CH), lambda i: (i, 0)),
        ],
        out_shape=[
            jax.ShapeDtypeStruct((NPAD, 1), jnp.float32),
            jax.ShapeDtypeStruct((NPAD, IN_CH), jnp.float32),
        ],
    )(hist, x_pad)


def _mm_body(acc0_ref, acc1_ref, y_ref, dinv_ref, w1_ref, b1_ref, w2_ref,
             y2_ref):
    agg = (acc0_ref[...] + acc1_ref[...] + y_ref[...]) * dinv_ref[...]
    h = jnp.dot(agg, w1_ref[...], preferred_element_type=jnp.float32)
    h = jnp.maximum(h + b1_ref[...], 0.0)
    hw = jnp.dot(h, w2_ref[...], preferred_element_type=jnp.float32)
    y2_ref[...] = hw * dinv_ref[...]


def _both_mm(acc, y1, dinv, W1, b1, W2):
    return pl.pallas_call(
        _mm_body,
        grid=(GRID_M,),
        in_specs=[
            pl.BlockSpec((MBLK, IN_CH), lambda i: (i, 0)),
            pl.BlockSpec((MBLK, IN_CH), lambda i: (GRID_M + i, 0)),
            pl.BlockSpec((MBLK, IN_CH), lambda i: (i, 0)),
            pl.BlockSpec((MBLK, 1), lambda i: (i, 0)),
            pl.BlockSpec((IN_CH, HID_CH), lambda i: (0, 0)),
            pl.BlockSpec((1, HID_CH), lambda i: (0, 0)),
            pl.BlockSpec((HID_CH, OUT_CH), lambda i: (0, 0)),
        ],
        out_specs=pl.BlockSpec((MBLK, OUT_CH), lambda i: (i, 0)),
        out_shape=jax.ShapeDtypeStruct((NPAD, OUT_CH), jnp.float32),
    )(acc, acc, y1, dinv, W1, b1, W2)


def _final_body(acc0_ref, acc1_ref, y2_ref, dinv_ref, b_ref, out_ref):
    agg = (acc0_ref[...] + acc1_ref[...] + y2_ref[...]) * dinv_ref[...]
    out_ref[...] = jnp.maximum(agg + b_ref[...], 0.0)


def _final_layer(acc, y2, dinv, b2):
    return pl.pallas_call(
        _final_body,
        grid=(GRID_M,),
        in_specs=[
            pl.BlockSpec((MBLK, OUT_CH), lambda i: (i, 0)),
            pl.BlockSpec((MBLK, OUT_CH), lambda i: (GRID_M + i, 0)),
            pl.BlockSpec((MBLK, OUT_CH), lambda i: (i, 0)),
            pl.BlockSpec((MBLK, 1), lambda i: (i, 0)),
            pl.BlockSpec((1, OUT_CH), lambda i: (0, 0)),
        ],
        out_specs=pl.BlockSpec((MBLK, OUT_CH), lambda i: (i, 0)),
        out_shape=jax.ShapeDtypeStruct((NPAD, OUT_CH), jnp.float32),
    )(acc, acc, y2, dinv, b2)


def kernel(x, edge_index, W1, b1, W2, b2):
    src = edge_index[0].astype(jnp.int32)
    dst = edge_index[1].astype(jnp.int32)
    src_p = src.reshape(NW, NPASS, CPP, CHUNK)
    dst_p = dst.reshape(NW, NPASS, CPP, CHUNK)
    dst_flat_tiles = dst.reshape(NW, E_TILE)
    x_pad = jnp.pad(x, ((0, NPAD - N_NODES), (0, 0)))
    b1r = b1.reshape(1, HID_CH)
    b2r = b2.reshape(1, OUT_CH)

    hist = _deg_partials(dst_flat_tiles)
    dinv, y1 = _dinv_and_y(hist, x_pad)

    acc1 = _edge_aggregate(y1, src_p, dst_p)
    y2 = _both_mm(acc1, y1, dinv, W1, b1r, W2)
    acc2 = _edge_aggregate(y2, src_p, dst_p)
    out = _final_layer(acc2, y2, dinv, b2r)
    return out[:N_NODES]
